# Initial kernel scaffold; baseline (speedup 1.0000x reference)
import jax
import jax.numpy as jnp
from jax.experimental import pallas as pl


def kernel(enc_hidden_states, dec_tokens, enc_attn_mask, dec_attn_mask, enc_dec_attn_mask, dec_labels, table):
    raise NotImplementedError("write your pallas kernel here")


# speedup vs baseline: 1.0304x; 1.0304x over previous
import functools

import jax
import jax.numpy as jnp
from jax import lax
from jax.experimental import pallas as pl
from jax.experimental.pallas import tpu as pltpu
from jax.experimental.pallas import tpu_sc as plsc


@functools.lru_cache(maxsize=None)
def _make_gather(N: int, V: int, D: int):
    info = plsc.get_sparse_core_info()
    NC, NS = info.num_cores, info.num_subcores
    NW = NC * NS
    assert N % NW == 0
    rows_per_w = N // NW
    CH = 64
    nchunk = rows_per_w // CH
    assert rows_per_w % CH == 0

    mesh = plsc.VectorSubcoreMesh(core_axis_name="c", subcore_axis_name="s")

    @functools.partial(
        pl.kernel,
        mesh=mesh,
        out_type=jax.ShapeDtypeStruct((N, D), jnp.float32),
        scratch_types=[
            pltpu.VMEM((rows_per_w,), jnp.int32),
            pltpu.VMEM((CH, D), jnp.float32),
            pltpu.SemaphoreType.DMA,
        ],
    )
    def gather_k(idx_hbm, table_hbm, out_hbm, idx_v, buf, gsem):
        wid = lax.axis_index("s") * NC + lax.axis_index("c")
        base = wid * rows_per_w
        pltpu.sync_copy(idx_hbm.at[pl.ds(base, rows_per_w)], idx_v)
        for c in range(nchunk):
            pltpu.async_copy(
                table_hbm.at[idx_v.at[pl.ds(c * CH, CH)]], buf, gsem
            ).wait()
            pltpu.sync_copy(buf, out_hbm.at[pl.ds(base + c * CH, CH)])

    return gather_k


def kernel(enc_hidden_states, dec_tokens, enc_attn_mask, dec_attn_mask,
           enc_dec_attn_mask, dec_labels, table):
    B, S = dec_tokens.shape
    V, D = table.shape
    idx = jnp.transpose(dec_tokens, (1, 0)).reshape(-1).astype(jnp.int32)
    flat = _make_gather(B * S, V, D)(idx, table)
    hidden_states = flat.reshape(S, B, D)
    return (enc_hidden_states, hidden_states)


# speedup vs baseline: 1.0320x; 1.0015x over previous
import functools

import jax
import jax.numpy as jnp
from jax import lax
from jax.experimental import pallas as pl
from jax.experimental.pallas import tpu as pltpu
from jax.experimental.pallas import tpu_sc as plsc


@functools.lru_cache(maxsize=None)
def _make_gather(N: int, V: int, D: int):
    info = plsc.get_sparse_core_info()
    NC, NS = info.num_cores, info.num_subcores
    NW = NC * NS
    assert N % NW == 0
    rows_per_w = N // NW
    CH = 32
    NB = 3
    nchunk = rows_per_w // CH
    assert rows_per_w % CH == 0

    mesh = plsc.VectorSubcoreMesh(core_axis_name="c", subcore_axis_name="s")

    @functools.partial(
        pl.kernel,
        mesh=mesh,
        out_type=jax.ShapeDtypeStruct((N, D), jnp.float32),
        scratch_types=[
            pltpu.VMEM((rows_per_w,), jnp.int32),
        ] + [pltpu.VMEM((CH, D), jnp.float32) for _ in range(NB)]
          + [pltpu.SemaphoreType.DMA for _ in range(2 * NB)],
    )
    def gather_k(idx_hbm, table_hbm, out_hbm, idx_v, *bufs_and_sems):
        bufs = bufs_and_sems[:NB]
        gsems = bufs_and_sems[NB:2 * NB]
        osems = bufs_and_sems[2 * NB:]
        wid = lax.axis_index("s") * NC + lax.axis_index("c")
        base = wid * rows_per_w
        pltpu.sync_copy(idx_hbm.at[pl.ds(base, rows_per_w)], idx_v)

        def start_gather(c):
            b = c % NB
            return pltpu.async_copy(
                table_hbm.at[idx_v.at[pl.ds(c * CH, CH)]], bufs[b], gsems[b])

        def start_outcopy(c):
            b = c % NB
            return pltpu.async_copy(
                bufs[b], out_hbm.at[pl.ds(base + c * CH, CH)], osems[b])

        ghandles = [None] * nchunk
        ohandles = [None] * nchunk
        owaited = [False] * nchunk
        for c in range(min(NB, nchunk)):
            ghandles[c] = start_gather(c)
        for c in range(nchunk):
            ghandles[c].wait()
            ohandles[c] = start_outcopy(c)
            nxt = c + NB - 1
            if c >= 1 and nxt < nchunk:
                ohandles[c - 1].wait()
                owaited[c - 1] = True
                ghandles[nxt] = start_gather(nxt)
        for c in range(nchunk):
            if not owaited[c]:
                ohandles[c].wait()

    return gather_k


def kernel(enc_hidden_states, dec_tokens, enc_attn_mask, dec_attn_mask,
           enc_dec_attn_mask, dec_labels, table):
    B, S = dec_tokens.shape
    V, D = table.shape
    idx = jnp.transpose(dec_tokens, (1, 0)).reshape(-1).astype(jnp.int32)
    flat = _make_gather(B * S, V, D)(idx, table)
    hidden_states = flat.reshape(S, B, D)
    return (enc_hidden_states, hidden_states)


# speedup vs baseline: 1.5814x; 1.5324x over previous
import functools

import jax
import jax.numpy as jnp
from jax import lax
from jax.experimental import pallas as pl
from jax.experimental.pallas import tpu as pltpu
from jax.experimental.pallas import tpu_sc as plsc


@functools.lru_cache(maxsize=None)
def _make_gather(S: int, B: int, V: int, D: int):
    info = plsc.get_sparse_core_info()
    NC, NS = info.num_cores, info.num_subcores
    NW = NC * NS
    N = S * B
    assert N % NW == 0
    rows_per_w = N // NW
    CH = 32
    NB = 3
    nchunk = rows_per_w // CH
    assert rows_per_w % CH == 0

    mesh = plsc.VectorSubcoreMesh(core_axis_name="c", subcore_axis_name="s")

    @functools.partial(
        pl.kernel,
        mesh=mesh,
        out_type=jax.ShapeDtypeStruct((S, B, D), jnp.float32),
        scratch_types=[
            pltpu.VMEM((rows_per_w,), jnp.int32),
        ] + [pltpu.VMEM((CH, D), jnp.float32) for _ in range(NB)]
          + [pltpu.SemaphoreType.DMA for _ in range(2 * NB)],
    )
    def gather_k(idx_hbm, table_hbm, out_hbm, idx_v, *bufs_and_sems):
        bufs = bufs_and_sems[:NB]
        gsems = bufs_and_sems[NB:2 * NB]
        osems = bufs_and_sems[2 * NB:]
        out_flat = out_hbm.reshape(N, D)
        wid = lax.axis_index("s") * NC + lax.axis_index("c")
        base = wid * rows_per_w
        pltpu.sync_copy(idx_hbm.at[pl.ds(base, rows_per_w)], idx_v)

        def start_gather(c):
            b = c % NB
            return pltpu.async_copy(
                table_hbm.at[idx_v.at[pl.ds(c * CH, CH)]], bufs[b], gsems[b])

        def start_outcopy(c):
            b = c % NB
            return pltpu.async_copy(
                bufs[b], out_flat.at[pl.ds(base + c * CH, CH)], osems[b])

        ghandles = [None] * nchunk
        ohandles = [None] * nchunk
        owaited = [False] * nchunk
        for c in range(min(NB, nchunk)):
            ghandles[c] = start_gather(c)
        for c in range(nchunk):
            ghandles[c].wait()
            ohandles[c] = start_outcopy(c)
            nxt = c + NB - 1
            if c >= 1 and nxt < nchunk:
                ohandles[c - 1].wait()
                owaited[c - 1] = True
                ghandles[nxt] = start_gather(nxt)
        for c in range(nchunk):
            if not owaited[c]:
                ohandles[c].wait()

    return gather_k


def kernel(enc_hidden_states, dec_tokens, enc_attn_mask, dec_attn_mask,
           enc_dec_attn_mask, dec_labels, table):
    B, S = dec_tokens.shape
    V, D = table.shape
    idx = jnp.transpose(dec_tokens, (1, 0)).reshape(-1).astype(jnp.int32)
    hidden_states = _make_gather(S, B, V, D)(idx, table)
    return (enc_hidden_states, hidden_states)


# speedup vs baseline: 1.6065x; 1.0159x over previous
import functools

import jax
import jax.numpy as jnp
from jax import lax
from jax.experimental import pallas as pl
from jax.experimental.pallas import tpu as pltpu
from jax.experimental.pallas import tpu_sc as plsc


@functools.lru_cache(maxsize=None)
def _make_kernel(S: int, B: int, V: int, D: int):
    info = plsc.get_sparse_core_info()
    NC, NS = info.num_cores, info.num_subcores
    NW = NC * NS
    N = S * B
    assert N % NW == 0
    rows_per_w = N // NW
    CH = 16
    NB = 3
    nchunk = rows_per_w // CH
    ECH = 32
    NBE = 2
    nechunk = rows_per_w // ECH
    assert rows_per_w % CH == 0 and rows_per_w % ECH == 0

    mesh = plsc.VectorSubcoreMesh(core_axis_name="c", subcore_axis_name="s")

    @functools.partial(
        pl.kernel,
        mesh=mesh,
        out_type=[
            jax.ShapeDtypeStruct((B, S, D), jnp.float32),
            jax.ShapeDtypeStruct((S, B, D), jnp.float32),
        ],
        scratch_types=[
            pltpu.VMEM((rows_per_w,), jnp.int32),
            pltpu.VMEM_SHARED((NS, NBE, ECH, D), jnp.float32),
        ] + [pltpu.VMEM((CH, D), jnp.float32) for _ in range(NB)]
          + [pltpu.SemaphoreType.DMA for _ in range(2 * NB + 2 * NBE)],
    )
    def gather_k(idx_hbm, table_hbm, enc_hbm, enc_out_hbm, out_hbm,
                 idx_v, ebuf_shared, *rest):
        gbufs = rest[:NB]
        sems = rest[NB:]
        gsems = sems[:NB]
        osems = sems[NB:2 * NB]
        eisems = sems[2 * NB:2 * NB + NBE]
        eosems = sems[2 * NB + NBE:]
        out_flat = out_hbm.reshape(N, D)
        enc_flat = enc_hbm.reshape(N, D)
        enc_out_flat = enc_out_hbm.reshape(N, D)
        sid = lax.axis_index("s")
        wid = sid * NC + lax.axis_index("c")
        base = wid * rows_per_w
        ebufs = [ebuf_shared.at[sid, b] for b in range(NBE)]
        pltpu.sync_copy(idx_hbm.at[pl.ds(base, rows_per_w)], idx_v)

        def g_start_in(c):
            b = c % NB
            return pltpu.async_copy(
                table_hbm.at[idx_v.at[pl.ds(c * CH, CH)]], gbufs[b], gsems[b])

        def g_start_out(c):
            b = c % NB
            return pltpu.async_copy(
                gbufs[b], out_flat.at[pl.ds(base + c * CH, CH)], osems[b])

        def e_start_in(c):
            b = c % NBE
            return pltpu.async_copy(
                enc_flat.at[pl.ds(base + c * ECH, ECH)], ebufs[b], eisems[b])

        def e_start_out(c):
            b = c % NBE
            return pltpu.async_copy(
                ebufs[b], enc_out_flat.at[pl.ds(base + c * ECH, ECH)],
                eosems[b])

        g_in = [None] * nchunk
        g_out = [None] * nchunk
        g_ow = [False] * nchunk
        e_in = [None] * nechunk
        e_out = [None] * nechunk
        e_ow = [False] * nechunk

        for c in range(min(NBE, nechunk)):
            e_in[c] = e_start_in(c)
        for c in range(min(NB, nchunk)):
            g_in[c] = g_start_in(c)

        def e_step(c):
            e_in[c].wait()
            e_out[c] = e_start_out(c)
            nxt = c + NBE - 1
            if c >= 1 and nxt < nechunk:
                e_out[c - 1].wait()
                e_ow[c - 1] = True
                e_in[nxt] = e_start_in(nxt)

        def g_step(c):
            g_in[c].wait()
            g_out[c] = g_start_out(c)
            nxt = c + NB - 1
            if c >= 1 and nxt < nchunk:
                g_out[c - 1].wait()
                g_ow[c - 1] = True
                g_in[nxt] = g_start_in(nxt)

        steps = max(nchunk, nechunk)
        for c in range(steps):
            if c < nchunk:
                g_step(c)
            if c < nechunk:
                e_step(c)

        for c in range(nchunk):
            if not g_ow[c]:
                g_out[c].wait()
        for c in range(nechunk):
            if not e_ow[c]:
                e_out[c].wait()

    return gather_k


def kernel(enc_hidden_states, dec_tokens, enc_attn_mask, dec_attn_mask,
           enc_dec_attn_mask, dec_labels, table):
    B, S = dec_tokens.shape
    V, D = table.shape
    idx = jnp.transpose(dec_tokens, (1, 0)).reshape(-1).astype(jnp.int32)
    enc_out, hidden_states = _make_kernel(S, B, V, D)(
        idx, table, enc_hidden_states)
    return (enc_out, hidden_states)


# speedup vs baseline: 1.6513x; 1.0278x over previous
import functools

import jax
import jax.numpy as jnp
from jax import lax
from jax.experimental import pallas as pl
from jax.experimental.pallas import tpu as pltpu
from jax.experimental.pallas import tpu_sc as plsc


@functools.lru_cache(maxsize=None)
def _make_gather(S: int, B: int, V: int, D: int):
    info = plsc.get_sparse_core_info()
    NC, NS = info.num_cores, info.num_subcores
    NW = NC * NS
    N = S * B
    assert N % NW == 0
    rows_per_w = N // NW
    CH = 32
    NB = 3
    nchunk = rows_per_w // CH
    assert rows_per_w % CH == 0

    mesh = plsc.VectorSubcoreMesh(core_axis_name="c", subcore_axis_name="s")

    @functools.partial(
        pl.kernel,
        mesh=mesh,
        out_type=jax.ShapeDtypeStruct((S, B, D), jnp.float32),
        scratch_types=[
            pltpu.VMEM((rows_per_w,), jnp.int32),
        ] + [pltpu.VMEM((CH, D), jnp.float32) for _ in range(NB)]
          + [pltpu.SemaphoreType.DMA for _ in range(2 * NB)],
    )
    def gather_k(idx_hbm, table_hbm, out_hbm, idx_v, *bufs_and_sems):
        bufs = bufs_and_sems[:NB]
        gsems = bufs_and_sems[NB:2 * NB]
        osems = bufs_and_sems[2 * NB:]
        out_flat = out_hbm.reshape(N, D)
        wid = lax.axis_index("s") * NC + lax.axis_index("c")
        base = wid * rows_per_w
        pltpu.sync_copy(idx_hbm.at[pl.ds(base, rows_per_w)], idx_v)

        def start_gather(c):
            b = c % NB
            return pltpu.async_copy(
                table_hbm.at[idx_v.at[pl.ds(c * CH, CH)]], bufs[b], gsems[b])

        def start_outcopy(c):
            b = c % NB
            return pltpu.async_copy(
                bufs[b], out_flat.at[pl.ds(base + c * CH, CH)], osems[b])

        ghandles = [None] * nchunk
        ohandles = [None] * nchunk
        owaited = [False] * nchunk
        for c in range(min(NB, nchunk)):
            ghandles[c] = start_gather(c)
        for c in range(nchunk):
            ghandles[c].wait()
            ohandles[c] = start_outcopy(c)
            nxt = c + NB - 1
            if c >= 1 and nxt < nchunk:
                ohandles[c - 1].wait()
                owaited[c - 1] = True
                ghandles[nxt] = start_gather(nxt)
        for c in range(nchunk):
            if not owaited[c]:
                ohandles[c].wait()

    return gather_k


def _copy_block(src_ref, dst_ref):
    dst_ref[...] = src_ref[...]


@functools.lru_cache(maxsize=None)
def _make_passthrough(B: int, S: int, D: int):
    BLK = 256
    N = B * S

    def run(x):
        flat = x.reshape(N, D)
        out = pl.pallas_call(
            _copy_block,
            grid=(N // BLK,),
            in_specs=[pl.BlockSpec((BLK, D), lambda i: (i, 0))],
            out_specs=pl.BlockSpec((BLK, D), lambda i: (i, 0)),
            out_shape=jax.ShapeDtypeStruct((N, D), jnp.float32),
        )(flat)
        return out.reshape(B, S, D)

    return run


def kernel(enc_hidden_states, dec_tokens, enc_attn_mask, dec_attn_mask,
           enc_dec_attn_mask, dec_labels, table):
    B, S = dec_tokens.shape
    V, D = table.shape
    idx = jnp.transpose(dec_tokens, (1, 0)).reshape(-1).astype(jnp.int32)
    hidden_states = _make_gather(S, B, V, D)(idx, table)
    enc_out = _make_passthrough(B, S, D)(enc_hidden_states)
    return (enc_out, hidden_states)
